# Initial kernel scaffold; baseline (speedup 1.0000x reference)
import functools

import jax
import jax.numpy as jnp
from jax import lax
from jax.experimental import pallas as pl
from jax.experimental.pallas import tpu as pltpu
from jax.experimental.pallas import tpu_sc as plsc

NC = 2
NS = 16
NW = NC * NS
K = 128


def _tree_norm_body(C, B, CPT, xT_hbm, ids_hbm, segs_hbm, start_hbm, s_hbm,
                    start_v, idx_v, seg_v, rows_v, acc, sem):
    wid = lax.axis_index("s") * NC + lax.axis_index("c")
    c0 = wid * CPT

    pltpu.sync_copy(start_hbm, start_v)
    p0 = start_v[wid]
    p1 = start_v[wid + 1]
    a0 = (p0 >> 3) << 3
    nch = (p1 - a0 + K - 1) // K

    inf16 = jnp.full((16,), jnp.inf, dtype=jnp.float32)

    def init_row(i, _):
        for k in range(8):
            acc[i, pl.ds(k * 16, 16)] = inf16
        return 0

    lax.fori_loop(0, CPT + 1, init_row, 0)

    def chunk(g, _):
        off = a0 + g * K
        pltpu.sync_copy(ids_hbm.at[pl.ds(off, K)], idx_v)
        pltpu.sync_copy(segs_hbm.at[pl.ds(off, K)], seg_v)
        pltpu.async_copy(xT_hbm.at[idx_v], rows_v, sem).wait()

        def pair(j, _):
            loc = seg_v[j] - c0
            valid = jnp.logical_and(loc >= 0, loc < CPT)
            row = jnp.where(valid, loc, CPT)
            for k in range(8):
                a = acc[row, pl.ds(k * 16, 16)]
                gv = rows_v[j, pl.ds(k * 16, 16)]
                acc[row, pl.ds(k * 16, 16)] = jnp.minimum(a, gv)
            return 0

        lax.fori_loop(0, K, pair, 0)
        return 0

    lax.fori_loop(0, nch, chunk, 0)
    pltpu.sync_copy(acc.at[pl.ds(0, CPT)], s_hbm.at[pl.ds(c0, CPT)])


def kernel(inputs, ids, segments):
    B, C = inputs.shape
    N = ids.shape[0]
    CPT = (C + NW - 1) // NW
    CP = NW * CPT

    xT = jnp.transpose(inputs)
    boundaries = jnp.arange(NW + 1, dtype=jnp.int32) * CPT
    start = jnp.searchsorted(segments, boundaries, side="left").astype(jnp.int32)
    start = jnp.concatenate([start, jnp.full((7,), N, jnp.int32)])
    ids_pad = jnp.concatenate([ids, jnp.zeros((K,), jnp.int32)])
    segs_pad = jnp.concatenate([segments, jnp.full((K,), C, jnp.int32)])

    mesh = plsc.VectorSubcoreMesh(core_axis_name="c", subcore_axis_name="s")
    body = functools.partial(_tree_norm_body, C, B, CPT)
    s = pl.kernel(
        body,
        out_type=jax.ShapeDtypeStruct((CP, B), jnp.float32),
        mesh=mesh,
        scratch_types=[
            pltpu.VMEM((NW + 8,), jnp.int32),
            pltpu.VMEM((K,), jnp.int32),
            pltpu.VMEM((K,), jnp.int32),
            pltpu.VMEM((K, B), jnp.float32),
            pltpu.VMEM((CPT + 1, B), jnp.float32),
            pltpu.SemaphoreType.DMA,
        ],
    )(xT, ids_pad, segs_pad, start)
    return jnp.transpose(s[:C])


# speedup vs baseline: 2.9319x; 2.9319x over previous
import functools

import jax
import jax.numpy as jnp
from jax import lax
from jax.experimental import pallas as pl
from jax.experimental.pallas import tpu as pltpu
from jax.experimental.pallas import tpu_sc as plsc

NC = 2
NS = 16
NW = NC * NS
K = 128


def _tree_norm_body(C, B, CPT, xT_hbm, ids_hbm, segs_hbm, start_hbm, s_hbm,
                    start_v, idx_v, seg_v, rows_v, acc, sem):
    wid = lax.axis_index("s") * NC + lax.axis_index("c")
    c0 = wid * CPT

    pltpu.sync_copy(start_hbm, start_v)
    pvec = start_v[pl.ds(wid, 16)]
    p0 = pvec[0]
    p1 = pvec[1]
    a0 = (p0 >> 3) << 3
    nch = (p1 - a0 + K - 1) // K

    inf16 = jnp.full((16,), jnp.inf, dtype=jnp.float32)

    def init_row(i, _):
        for k in range(8):
            acc[i, pl.ds(k * 16, 16)] = inf16
        return 0

    lax.fori_loop(0, CPT + 1, init_row, 0)

    def chunk(g, _):
        off = pl.multiple_of(a0 + g * K, 8)
        pltpu.sync_copy(ids_hbm.at[pl.ds(off, K)], idx_v)
        pltpu.sync_copy(segs_hbm.at[pl.ds(off, K)], seg_v)
        pltpu.async_copy(xT_hbm.at[idx_v], rows_v, sem).wait()

        def pair16(jj, _):
            segvec = seg_v[pl.ds(jj * 16, 16)]
            for t in range(16):
                j = jj * 16 + t
                loc = segvec[t] - c0
                valid = jnp.logical_and(loc >= 0, loc < CPT)
                row = jnp.where(valid, loc, CPT)
                for k in range(8):
                    a = acc[row, pl.ds(k * 16, 16)]
                    gv = rows_v[j, pl.ds(k * 16, 16)]
                    acc[row, pl.ds(k * 16, 16)] = jnp.minimum(a, gv)
            return 0

        lax.fori_loop(0, K // 16, pair16, 0)
        return 0

    lax.fori_loop(0, nch, chunk, 0)
    pltpu.sync_copy(acc.at[pl.ds(0, CPT)], s_hbm.at[pl.ds(c0, CPT)])


def kernel(inputs, ids, segments):
    B, C = inputs.shape
    N = ids.shape[0]
    CPT = (-(-C // NW) + 7) // 8 * 8
    CP = NW * CPT

    xT = jnp.transpose(inputs)
    boundaries = jnp.arange(NW + 1, dtype=jnp.int32) * CPT
    start = jnp.searchsorted(segments, boundaries, side="left").astype(jnp.int32)
    start = jnp.concatenate([start, jnp.full((15,), N, jnp.int32)])
    ids_pad = jnp.concatenate([ids, jnp.zeros((K,), jnp.int32)])
    segs_pad = jnp.concatenate([segments, jnp.full((K,), C, jnp.int32)])

    mesh = plsc.VectorSubcoreMesh(
        core_axis_name="c", subcore_axis_name="s", num_cores=NC, num_subcores=NS
    )
    body = functools.partial(_tree_norm_body, C, B, CPT)
    s = pl.kernel(
        body,
        out_type=jax.ShapeDtypeStruct((CP, B), jnp.float32),
        mesh=mesh,
        scratch_types=[
            pltpu.VMEM((NW + 16,), jnp.int32),
            pltpu.VMEM((K,), jnp.int32),
            pltpu.VMEM((K,), jnp.int32),
            pltpu.VMEM((K, B), jnp.float32),
            pltpu.VMEM((CPT + 1, B), jnp.float32),
            pltpu.SemaphoreType.DMA,
        ],
    )(xT, ids_pad, segs_pad, start)
    return jnp.transpose(s[:C])
